# baseline (speedup 1.0000x reference)
import functools

import jax
import jax.numpy as jnp
from jax import lax
from jax.experimental import pallas as pl
from jax.experimental.pallas import tpu as pltpu
from jax.experimental.pallas import tpu_sc as plsc

_HASH_MOD = 1000001
_LANES = 16
_PHASES = 16
_PAD = 32


@functools.cache
def _build(xshape: tuple, dim: int):
    n_rows, n_cols = xshape
    info = plsc.get_sparse_core_info()
    nc, ns = info.num_cores, info.num_subcores
    nw = nc * ns
    assert n_rows % (nw * _PHASES) == 0
    assert _LANES <= n_cols <= _PAD
    rows_w = n_rows // nw
    rows_p = rows_w // _PHASES
    mesh = plsc.VectorSubcoreMesh(core_axis_name="c", subcore_axis_name="s")

    @functools.partial(
        pl.kernel,
        out_type=jax.ShapeDtypeStruct((n_rows, n_cols, dim), jnp.float32),
        mesh=mesh,
        compiler_params=pltpu.CompilerParams(use_tc_tiling_on_sc=False),
        scratch_types=[
            pltpu.VMEM((rows_w, n_cols), jnp.int32),
            pltpu.VMEM((rows_w, _PAD), jnp.int32),
            pltpu.VMEM((2, rows_p, n_cols, dim), jnp.float32),
            pltpu.SemaphoreType.DMA,
            pltpu.SemaphoreType.DMA,
            pltpu.SemaphoreType.DMA,
        ],
    )
    def k(x_hbm, table_hbm, out_hbm, idx_v, hidx_v, rows_v, gsem, osem0, osem1):
        osem = (osem0, osem1)
        wid = lax.axis_index("s") * nc + lax.axis_index("c")
        r0 = wid * rows_w
        pltpu.sync_copy(x_hbm.at[pl.ds(r0, rows_w)], idx_v)

        def mod_phase(p):
            def body(i, carry):
                r = p * rows_p + i
                va = idx_v[r, pl.ds(0, _LANES)]
                hidx_v[r, pl.ds(0, _LANES)] = lax.rem(
                    va, lax.full_like(va, _HASH_MOD)
                )
                vb = idx_v[r, pl.ds(n_cols - _LANES, _LANES)]
                hidx_v[r, pl.ds(n_cols - _LANES, _LANES)] = lax.rem(
                    vb, lax.full_like(vb, _HASH_MOD)
                )
                return carry

            lax.fori_loop(0, rows_p, body, 0)

        def row_gather(p, b, i):
            return pltpu.make_async_copy(
                table_hbm.at[hidx_v.at[p * rows_p + i].at[pl.ds(0, n_cols)]],
                rows_v.at[b, i],
                gsem,
            )

        def gather_start(p, b):
            lax.fori_loop(
                0, rows_p, lambda i, c: (row_gather(p, b, i).start(), c)[1], 0
            )

        def gather_wait(p, b):
            lax.fori_loop(
                0, rows_p, lambda i, c: (row_gather(p, b, i).wait(), c)[1], 0
            )

        def write_copy(p, b):
            return pltpu.make_async_copy(
                rows_v.at[b],
                out_hbm.at[pl.ds(r0 + p * rows_p, rows_p)],
                osem[b],
            )

        mod_phase(0)
        gather_start(0, 0)
        for p in range(_PHASES):
            b = p % 2
            if p + 1 < _PHASES:
                mod_phase(p + 1)
                gather_wait(p, b)
                if p >= 1:
                    write_copy(p - 1, 1 - b).wait()
                gather_start(p + 1, 1 - b)
            else:
                gather_wait(p, b)
            write_copy(p, b).start()
        write_copy(_PHASES - 2, _PHASES % 2).wait()
        write_copy(_PHASES - 1, (_PHASES - 1) % 2).wait()

    return k


def kernel(x, table):
    return _build(x.shape, table.shape[1])(x, table)
